# baseline (speedup 1.0000x reference)
import functools

import jax
import jax.numpy as jnp
from jax import lax
from jax.experimental import pallas as pl
from jax.experimental.pallas import tpu as pltpu
from jax.experimental.pallas import tpu_sc as plsc

N = 10000
NPAD = 10240
PT = NPAD // 16
NPS = 10112
PTS = NPS // 16
E = 320000
F = 128
NC, NS = 2, 16
NW = NC * NS
CS = 128
CPW = 80
EPAD = NW * CPW * CS
PADE = EPAD - E
ECH = EPAD // CS

_mesh = plsc.VectorSubcoreMesh(core_axis_name="c", subcore_axis_name="s")

CPC = EPAD // (NS * CS)


@functools.partial(
    pl.kernel,
    out_type=[
        jax.ShapeDtypeStruct((NPAD,), jnp.float32),
        jax.ShapeDtypeStruct((NPAD,), jnp.float32),
    ],
    mesh=_mesh,
    scratch_types=[
        pltpu.VMEM_SHARED((NPAD,), jnp.float32),
        pltpu.VMEM((CPC, CS), jnp.int32),
        pltpu.VMEM((CS,), jnp.float32),
    ],
)
def _sc_degrees(s_hbm, d_hbm, z_hbm, outs_hbm, outd_hbm, acc, idx, ones_v):
    c = lax.axis_index("c")
    t = lax.axis_index("s")
    for i in range(CS // 16):
        ones_v[pl.ds(i * 16, 16)] = jnp.ones((16,), jnp.float32)
    zb = t * PT
    pltpu.sync_copy(z_hbm.at[pl.ds(zb, PT)], acc.at[pl.ds(zb, PT)])
    cb = t * CPC

    @pl.when(c == 0)
    def _():
        pltpu.sync_copy(s_hbm.at[pl.ds(cb, CPC), :], idx)

    @pl.when(c == 1)
    def _():
        pltpu.sync_copy(d_hbm.at[pl.ds(cb, CPC), :], idx)

    plsc.subcore_barrier()

    def body(i, carry):
        pltpu.sync_copy(ones_v, acc.at[idx.at[i]], add=True)
        return carry

    lax.fori_loop(0, CPC, body, 0)
    plsc.subcore_barrier()

    @pl.when(c == 0)
    def _():
        pltpu.sync_copy(acc.at[pl.ds(zb, PT)], outs_hbm.at[pl.ds(zb, PT)])

    @pl.when(c == 1)
    def _():
        pltpu.sync_copy(acc.at[pl.ds(zb, PT)], outd_hbm.at[pl.ds(zb, PT)])


NG = (CPW - 2) // 3


def _make_scatter(k):
    @functools.partial(
        pl.kernel,
        out_type=jax.ShapeDtypeStruct((NC, NPS, k), jnp.float32),
        mesh=_mesh,
        scratch_types=[
            pltpu.VMEM_SHARED((NPS, k), jnp.float32),
            pltpu.VMEM((6, CS), jnp.int32),
            pltpu.VMEM((3, CS, k), jnp.float32),
            pltpu.SemaphoreType.DMA,
            pltpu.SemaphoreType.DMA,
            pltpu.SemaphoreType.DMA,
        ],
    )
    def _sc_scatter(tab_hbm, gs_hbm, z_hbm, out_hbm,
                    acc, ix, rows, sem0, sem1, sem2):
        c = lax.axis_index("c")
        t = lax.axis_index("s")
        zb = t * PTS
        pltpu.sync_copy(z_hbm.at[pl.ds(zb, PTS), :], acc.at[pl.ds(zb, PTS), :])
        wst = (c * NS + t) * CPW
        plsc.subcore_barrier()

        pltpu.sync_copy(gs_hbm.at[wst], ix.at[pl.ds(0, 2), :])
        pltpu.async_copy(tab_hbm.at[ix.at[0]], rows.at[0], sem0)
        pltpu.sync_copy(gs_hbm.at[wst + 1], ix.at[pl.ds(2, 2), :])
        pltpu.async_copy(tab_hbm.at[ix.at[2]], rows.at[1], sem1)

        def group(j, carry):
            base = wst + j * 3
            pltpu.sync_copy(gs_hbm.at[base + 2], ix.at[pl.ds(4, 2), :])
            pltpu.async_copy(tab_hbm.at[ix.at[4]], rows.at[2], sem2)
            pltpu.make_async_copy(tab_hbm.at[ix.at[0]], rows.at[0], sem0).wait()
            pltpu.sync_copy(rows.at[0], acc.at[ix.at[1]], add=True)
            pltpu.sync_copy(gs_hbm.at[base + 3], ix.at[pl.ds(0, 2), :])
            pltpu.async_copy(tab_hbm.at[ix.at[0]], rows.at[0], sem0)
            pltpu.make_async_copy(tab_hbm.at[ix.at[2]], rows.at[1], sem1).wait()
            pltpu.sync_copy(rows.at[1], acc.at[ix.at[3]], add=True)
            pltpu.sync_copy(gs_hbm.at[base + 4], ix.at[pl.ds(2, 2), :])
            pltpu.async_copy(tab_hbm.at[ix.at[2]], rows.at[1], sem1)
            pltpu.make_async_copy(tab_hbm.at[ix.at[4]], rows.at[2], sem2).wait()
            pltpu.sync_copy(rows.at[2], acc.at[ix.at[5]], add=True)
            return carry

        lax.fori_loop(0, NG, group, 0)
        pltpu.make_async_copy(tab_hbm.at[ix.at[0]], rows.at[0], sem0).wait()
        pltpu.sync_copy(rows.at[0], acc.at[ix.at[1]], add=True)
        pltpu.make_async_copy(tab_hbm.at[ix.at[2]], rows.at[1], sem1).wait()
        pltpu.sync_copy(rows.at[1], acc.at[ix.at[3]], add=True)
        plsc.subcore_barrier()
        pltpu.sync_copy(acc.at[pl.ds(zb, PTS), :], out_hbm.at[c, pl.ds(zb, PTS), :])

    return _sc_scatter


_scatter128 = _make_scatter(128)


def _prep_body(deg_s, deg_d, x, ds_s_o, ds_d_o, xs_o):
    rs_s = lax.rsqrt(jnp.maximum(deg_s[...], 1.0))
    rs_d = lax.rsqrt(jnp.maximum(deg_d[...], 1.0))
    ds_s_o[...] = rs_s
    ds_d_o[...] = rs_d
    xs_o[0:N, :] = x[...] * rs_d
    xs_o[N:NPS, :] = jnp.zeros((NPS - N, F), jnp.float32)


_tc_prep = pl.pallas_call(
    _prep_body,
    out_shape=[
        jax.ShapeDtypeStruct((N, 1), jnp.float32),
        jax.ShapeDtypeStruct((N, 1), jnp.float32),
        jax.ShapeDtypeStruct((NPS, F), jnp.float32),
    ],
)


def _layer0_body(x, bp, ds_s, wb0, w0, o):
    ab = (bp[0, 0:N, :] + bp[1, 0:N, :]) * ds_s[...]
    hb = jnp.dot(ab, wb0[...], preferred_element_type=jnp.float32)
    h1 = jnp.maximum(x[...], hb)
    o[0:N, :] = jnp.dot(h1 * ds_s[...], w0[...], preferred_element_type=jnp.float32)
    o[N:NPS, :] = jnp.zeros((NPS - N, F), jnp.float32)


_tc_layer0 = pl.pallas_call(
    _layer0_body,
    out_shape=jax.ShapeDtypeStruct((NPS, F), jnp.float32),
)

BN_BN = 2000
BN_NB = 6


def _bnlayer_body(fp, bp, ds_s, ds_d, g, beta, bback, wb, w, o, sums, sumsq):
    p = pl.program_id(0)
    j = pl.program_id(1)
    rows = lax.broadcasted_iota(jnp.int32, (BN_BN, 1), 0) + j * BN_BN
    mask = rows < N
    h = (fp[0] + fp[1]) * ds_d[...]

    @pl.when(p == 0)
    def _():
        hm = jnp.where(mask, h, 0.0)

        @pl.when(j == 0)
        def _():
            sums[...] = jnp.zeros_like(sums)
            sumsq[...] = jnp.zeros_like(sumsq)

        sums[...] += jnp.sum(hm, axis=0, keepdims=True)
        sumsq[...] += jnp.sum(hm * hm, axis=0, keepdims=True)
        o[...] = jnp.zeros_like(o)

    @pl.when(p == 1)
    def _():
        m = sums[...] / N
        v = sumsq[...] / N - m * m
        inv = lax.rsqrt(v + 1e-5)
        h1 = jnp.maximum((h - m) * inv * g[...] + beta[...], 0.0)
        ab = (bp[0] + bp[1]) * ds_s[...]
        hbk = jnp.dot(ab, wb[...], preferred_element_type=jnp.float32) + bback[...]
        h2 = jnp.where(mask, jnp.maximum(h1, hbk) * ds_s[...], 0.0)
        o[...] = jnp.dot(h2, w[...], preferred_element_type=jnp.float32)


def _make_bnlayer(ko):
    return pl.pallas_call(
        _bnlayer_body,
        grid=(2, BN_NB),
        in_specs=[
            pl.BlockSpec((NC, BN_BN, F), lambda p, j: (0, j, 0)),
            pl.BlockSpec((NC, BN_BN, F), lambda p, j: (0, j, 0)),
            pl.BlockSpec((BN_BN, 1), lambda p, j: (j, 0)),
            pl.BlockSpec((BN_BN, 1), lambda p, j: (j, 0)),
            pl.BlockSpec((1, F), lambda p, j: (0, 0)),
            pl.BlockSpec((1, F), lambda p, j: (0, 0)),
            pl.BlockSpec((1, F), lambda p, j: (0, 0)),
            pl.BlockSpec((F, F), lambda p, j: (0, 0)),
            pl.BlockSpec((F, ko), lambda p, j: (0, 0)),
        ],
        out_specs=pl.BlockSpec((BN_BN, ko), lambda p, j: (j, 0)),
        out_shape=jax.ShapeDtypeStruct((NPS, ko), jnp.float32),
        scratch_shapes=[
            pltpu.VMEM((1, F), jnp.float32),
            pltpu.VMEM((1, F), jnp.float32),
        ],
    )


_tc_bnlayer128 = _make_bnlayer(128)


def _final_body(f2p, ds_d, b2p, o):
    o[...] = (f2p[0, 0:N, :] + f2p[1, 0:N, :]) * ds_d[...] + b2p[...]


_tc_final = pl.pallas_call(
    _final_body,
    out_shape=jax.ShapeDtypeStruct((N, 128), jnp.float32),
)


def kernel(x, edge_index, Wb0, Wb1, Wb2, bb2, W0, W1, W2, b2, g0, beta0, g1, beta1):
    s = edge_index[0]
    d = edge_index[1]
    pad = (jnp.arange(PADE, dtype=jnp.int32) % 112) + N
    sp = jnp.concatenate([s, pad]).reshape(ECH, CS)
    dp = jnp.concatenate([d, pad]).reshape(ECH, CS)
    gs_b = jnp.stack([dp, sp], axis=1)
    gs_f = jnp.stack([sp, dp], axis=1)
    z1 = jnp.zeros((NPAD,), jnp.float32)
    z128 = jnp.zeros((NPS, 128), jnp.float32)
    w2p = jnp.pad(W2, ((0, 0), (0, 88)))
    b2p = jnp.pad(b2, (0, 88))[None, :]
    zb = jnp.zeros((1, F), jnp.float32)

    deg_s, deg_d = _sc_degrees(sp, dp, z1)
    ds_s, ds_d, xs = _tc_prep(deg_s[:N, None], deg_d[:N, None], x)
    bagg = _scatter128(xs, gs_b, z128)
    tf0 = _tc_layer0(x, bagg, ds_s, Wb0, W0)
    f0 = _scatter128(tf0, gs_f, z128)
    tf1 = _tc_bnlayer128(f0, bagg, ds_s, ds_d, g0[None, :], beta0[None, :], zb, Wb1, W1)
    f1 = _scatter128(tf1, gs_f, z128)
    tf2 = _tc_bnlayer128(f1, bagg, ds_s, ds_d, g1[None, :], beta1[None, :], bb2[None, :], Wb2, w2p)
    f2 = _scatter128(tf2, gs_f, z128)
    out = _tc_final(f2, ds_d, b2p)
    return out[:, :40]


# speedup vs baseline: 1.0637x; 1.0637x over previous
import functools

import jax
import jax.numpy as jnp
from jax import lax
from jax.experimental import pallas as pl
from jax.experimental.pallas import tpu as pltpu
from jax.experimental.pallas import tpu_sc as plsc

N = 10000
NPAD = 10240
PT = NPAD // 16
E = 320000
F = 128
NC, NS = 2, 16
NW = NC * NS
CS = 128
CPW = 80
EPAD = NW * CPW * CS
PADE = EPAD - E
ECH = EPAD // CS

_mesh = plsc.VectorSubcoreMesh(core_axis_name="c", subcore_axis_name="s")

CPC = EPAD // (NS * CS)


@functools.partial(
    pl.kernel,
    out_type=[
        jax.ShapeDtypeStruct((NPAD,), jnp.float32),
        jax.ShapeDtypeStruct((NPAD,), jnp.float32),
    ],
    mesh=_mesh,
    scratch_types=[
        pltpu.VMEM_SHARED((NPAD,), jnp.float32),
        pltpu.VMEM((CPC, CS), jnp.int32),
        pltpu.VMEM((CS,), jnp.float32),
    ],
)
def _sc_degrees(s_hbm, d_hbm, z_hbm, outs_hbm, outd_hbm, acc, idx, ones_v):
    c = lax.axis_index("c")
    t = lax.axis_index("s")
    for i in range(CS // 16):
        ones_v[pl.ds(i * 16, 16)] = jnp.ones((16,), jnp.float32)
    zb = t * PT
    pltpu.sync_copy(z_hbm.at[pl.ds(zb, PT)], acc.at[pl.ds(zb, PT)])
    cb = t * CPC

    @pl.when(c == 0)
    def _():
        pltpu.sync_copy(s_hbm.at[pl.ds(cb, CPC), :], idx)

    @pl.when(c == 1)
    def _():
        pltpu.sync_copy(d_hbm.at[pl.ds(cb, CPC), :], idx)

    plsc.subcore_barrier()

    def body(i, carry):
        pltpu.sync_copy(ones_v, acc.at[idx.at[i]], add=True)
        return carry

    lax.fori_loop(0, CPC, body, 0)
    plsc.subcore_barrier()

    @pl.when(c == 0)
    def _():
        pltpu.sync_copy(acc.at[pl.ds(zb, PT)], outs_hbm.at[pl.ds(zb, PT)])

    @pl.when(c == 1)
    def _():
        pltpu.sync_copy(acc.at[pl.ds(zb, PT)], outd_hbm.at[pl.ds(zb, PT)])


def _make_scatter(k):
    @functools.partial(
        pl.kernel,
        out_type=jax.ShapeDtypeStruct((NC, NPAD, k), jnp.float32),
        mesh=_mesh,
        scratch_types=[
            pltpu.VMEM_SHARED((NPAD, k), jnp.float32),
            pltpu.VMEM((4, 2, CS), jnp.int32),
            pltpu.VMEM((CPW, CS), jnp.int32),
            pltpu.VMEM((2, CS, k), jnp.float32),
            pltpu.SemaphoreType.DMA,
            pltpu.SemaphoreType.DMA,
            pltpu.SemaphoreType.DMA,
            pltpu.SemaphoreType.DMA,
            pltpu.SemaphoreType.DMA,
            pltpu.SemaphoreType.DMA,
        ],
    )
    def _sc_scatter(tab_hbm, g_hbm, s_hbm, z_hbm, out_hbm,
                    acc, idx_g, idx_s, rows, sem0, sem1, si0, si1, si2, si3):
        c = lax.axis_index("c")
        t = lax.axis_index("s")
        zb = t * PT
        pltpu.sync_copy(z_hbm.at[pl.ds(zb, PT), :], acc.at[pl.ds(zb, PT), :])
        wst = (c * NS + t) * CPW
        pltpu.sync_copy(s_hbm.at[pl.ds(wst, CPW), :], idx_s)
        plsc.subcore_barrier()

        si = [si0, si1, si2, si3]

        pltpu.sync_copy(g_hbm.at[pl.ds(wst, 2), :], idx_g.at[3])
        pltpu.async_copy(tab_hbm.at[idx_g.at[3, 0]], rows.at[0], sem0)
        pltpu.async_copy(tab_hbm.at[idx_g.at[3, 1]], rows.at[1], sem1)
        pltpu.async_copy(g_hbm.at[pl.ds(wst + 2, 2), :], idx_g.at[0], si0)
        pltpu.async_copy(g_hbm.at[pl.ds(wst + 4, 2), :], idx_g.at[1], si1)
        pltpu.async_copy(g_hbm.at[pl.ds(wst + 6, 2), :], idx_g.at[2], si2)

        def _emit_pair(g, m, fire=True, load=True):
            if fire:
                pltpu.make_async_copy(g_hbm.at[pl.ds(wst + g + 2, 2), :],
                                      idx_g.at[m], si[m]).wait()
            pltpu.make_async_copy(tab_hbm.at[idx_g.at[0, 0]], rows.at[0], sem0).wait()
            pltpu.sync_copy(rows.at[0], acc.at[idx_s.at[g]], add=True)
            if fire:
                pltpu.async_copy(tab_hbm.at[idx_g.at[m, 0]], rows.at[0], sem0)
            pltpu.make_async_copy(tab_hbm.at[idx_g.at[0, 1]], rows.at[1], sem1).wait()
            pltpu.sync_copy(rows.at[1], acc.at[idx_s.at[g + 1]], add=True)
            if fire:
                pltpu.async_copy(tab_hbm.at[idx_g.at[m, 1]], rows.at[1], sem1)
            if load:
                pltpu.async_copy(g_hbm.at[pl.ds(wst + g + 8, 2), :],
                                 idx_g.at[(m + 3) % 4], si[(m + 3) % 4])

        def octet(r, carry):
            g = r * 8
            for pi in range(4):
                _emit_pair(g + 2 * pi, pi)
            return carry

        lax.fori_loop(0, (CPW - 8) // 8, octet, 0)
        _emit_pair(CPW - 8, 0, load=False)
        _emit_pair(CPW - 6, 1, load=False)
        _emit_pair(CPW - 4, 2, load=False)
        _emit_pair(CPW - 2, 3, fire=False, load=False)
        plsc.subcore_barrier()
        pltpu.sync_copy(acc.at[pl.ds(zb, PT), :], out_hbm.at[c, pl.ds(zb, PT), :])

    return _sc_scatter


_scatter128 = _make_scatter(128)


def _prep_body(deg_s, deg_d, x, ds_s_o, ds_d_o, xs_o):
    rs_s = lax.rsqrt(jnp.maximum(deg_s[...], 1.0))
    rs_d = lax.rsqrt(jnp.maximum(deg_d[...], 1.0))
    ds_s_o[...] = rs_s
    ds_d_o[...] = rs_d
    xs_o[0:N, :] = x[...] * rs_d
    xs_o[N:NPAD, :] = jnp.zeros((NPAD - N, F), jnp.float32)


_tc_prep = pl.pallas_call(
    _prep_body,
    out_shape=[
        jax.ShapeDtypeStruct((N, 1), jnp.float32),
        jax.ShapeDtypeStruct((N, 1), jnp.float32),
        jax.ShapeDtypeStruct((NPAD, F), jnp.float32),
    ],
)


def _layer0_body(x, bp, ds_s, wb0, w0, o):
    ab = (bp[0, 0:N, :] + bp[1, 0:N, :]) * ds_s[...]
    hb = jnp.dot(ab, wb0[...], preferred_element_type=jnp.float32)
    h1 = jnp.maximum(x[...], hb)
    o[0:N, :] = jnp.dot(h1 * ds_s[...], w0[...], preferred_element_type=jnp.float32)
    o[N:NPAD, :] = jnp.zeros((NPAD - N, F), jnp.float32)


_tc_layer0 = pl.pallas_call(
    _layer0_body,
    out_shape=jax.ShapeDtypeStruct((NPAD, F), jnp.float32),
)

BN_BN = 2000
BN_NB = 6


def _bnlayer_body(fp, bp, ds_s, ds_d, g, beta, bback, wb, w, o, sums, sumsq):
    p = pl.program_id(0)
    j = pl.program_id(1)
    rows = lax.broadcasted_iota(jnp.int32, (BN_BN, 1), 0) + j * BN_BN
    mask = rows < N
    h = (fp[0] + fp[1]) * ds_d[...]

    @pl.when(p == 0)
    def _():
        hm = jnp.where(mask, h, 0.0)

        @pl.when(j == 0)
        def _():
            sums[...] = jnp.zeros_like(sums)
            sumsq[...] = jnp.zeros_like(sumsq)

        sums[...] += jnp.sum(hm, axis=0, keepdims=True)
        sumsq[...] += jnp.sum(hm * hm, axis=0, keepdims=True)
        o[...] = jnp.zeros_like(o)

    @pl.when(p == 1)
    def _():
        m = sums[...] / N
        v = sumsq[...] / N - m * m
        inv = lax.rsqrt(v + 1e-5)
        h1 = jnp.maximum((h - m) * inv * g[...] + beta[...], 0.0)
        ab = (bp[0] + bp[1]) * ds_s[...]
        hbk = jnp.dot(ab, wb[...], preferred_element_type=jnp.float32) + bback[...]
        h2 = jnp.where(mask, jnp.maximum(h1, hbk) * ds_s[...], 0.0)
        o[...] = jnp.dot(h2, w[...], preferred_element_type=jnp.float32)


def _make_bnlayer(ko):
    return pl.pallas_call(
        _bnlayer_body,
        grid=(2, BN_NB),
        in_specs=[
            pl.BlockSpec((NC, BN_BN, F), lambda p, j: (0, j, 0)),
            pl.BlockSpec((NC, BN_BN, F), lambda p, j: (0, j, 0)),
            pl.BlockSpec((BN_BN, 1), lambda p, j: (j, 0)),
            pl.BlockSpec((BN_BN, 1), lambda p, j: (j, 0)),
            pl.BlockSpec((1, F), lambda p, j: (0, 0)),
            pl.BlockSpec((1, F), lambda p, j: (0, 0)),
            pl.BlockSpec((1, F), lambda p, j: (0, 0)),
            pl.BlockSpec((F, F), lambda p, j: (0, 0)),
            pl.BlockSpec((F, ko), lambda p, j: (0, 0)),
        ],
        out_specs=pl.BlockSpec((BN_BN, ko), lambda p, j: (j, 0)),
        out_shape=jax.ShapeDtypeStruct((NPAD, ko), jnp.float32),
        scratch_shapes=[
            pltpu.VMEM((1, F), jnp.float32),
            pltpu.VMEM((1, F), jnp.float32),
        ],
    )


_tc_bnlayer128 = _make_bnlayer(128)


def _final_body(f2p, ds_d, b2p, o):
    o[...] = (f2p[0, 0:N, :] + f2p[1, 0:N, :]) * ds_d[...] + b2p[...]


_tc_final = pl.pallas_call(
    _final_body,
    out_shape=jax.ShapeDtypeStruct((N, 128), jnp.float32),
)


def kernel(x, edge_index, Wb0, Wb1, Wb2, bb2, W0, W1, W2, b2, g0, beta0, g1, beta1):
    s = edge_index[0]
    d = edge_index[1]
    pad = (jnp.arange(PADE, dtype=jnp.int32) % 128) + N
    sp = jnp.concatenate([s, pad]).reshape(ECH, CS)
    dp = jnp.concatenate([d, pad]).reshape(ECH, CS)
    z1 = jnp.zeros((NPAD,), jnp.float32)
    z128 = jnp.zeros((NPAD, 128), jnp.float32)
    w2p = jnp.pad(W2, ((0, 0), (0, 88)))
    b2p = jnp.pad(b2, (0, 88))[None, :]
    zb = jnp.zeros((1, F), jnp.float32)

    deg_s, deg_d = _sc_degrees(sp, dp, z1)
    ds_s, ds_d, xs = _tc_prep(deg_s[:N, None], deg_d[:N, None], x)
    bagg = _scatter128(xs, dp, sp, z128)
    tf0 = _tc_layer0(x, bagg, ds_s, Wb0, W0)
    f0 = _scatter128(tf0, sp, dp, z128)
    tf1 = _tc_bnlayer128(f0, bagg, ds_s, ds_d, g0[None, :], beta0[None, :], zb, Wb1, W1)
    f1 = _scatter128(tf1, sp, dp, z128)
    tf2 = _tc_bnlayer128(f1, bagg, ds_s, ds_d, g1[None, :], beta1[None, :], bb2[None, :], Wb2, w2p)
    f2 = _scatter128(tf2, sp, dp, z128)
    out = _tc_final(f2, ds_d, b2p)
    return out[:, :40]


# speedup vs baseline: 1.0690x; 1.0049x over previous
import functools

import jax
import jax.numpy as jnp
from jax import lax
from jax.experimental import pallas as pl
from jax.experimental.pallas import tpu as pltpu
from jax.experimental.pallas import tpu_sc as plsc

N = 10000
NPAD = 10240
PT = NPAD // 16
E = 320000
F = 128
NC, NS = 2, 16
NW = NC * NS
CS = 128
CPW = 80
EPAD = NW * CPW * CS
PADE = EPAD - E
ECH = EPAD // CS

_mesh = plsc.VectorSubcoreMesh(core_axis_name="c", subcore_axis_name="s")

CPC = EPAD // (NS * CS)


@functools.partial(
    pl.kernel,
    out_type=[
        jax.ShapeDtypeStruct((NPAD,), jnp.float32),
        jax.ShapeDtypeStruct((NPAD,), jnp.float32),
    ],
    mesh=_mesh,
    scratch_types=[
        pltpu.VMEM_SHARED((NPAD,), jnp.float32),
        pltpu.VMEM((CPC, CS), jnp.int32),
        pltpu.VMEM((CS,), jnp.float32),
    ],
)
def _sc_degrees(s_hbm, d_hbm, z_hbm, outs_hbm, outd_hbm, acc, idx, ones_v):
    c = lax.axis_index("c")
    t = lax.axis_index("s")
    for i in range(CS // 16):
        ones_v[pl.ds(i * 16, 16)] = jnp.ones((16,), jnp.float32)
    zb = t * PT
    pltpu.sync_copy(z_hbm.at[pl.ds(zb, PT)], acc.at[pl.ds(zb, PT)])
    cb = t * CPC

    @pl.when(c == 0)
    def _():
        pltpu.sync_copy(s_hbm.at[pl.ds(cb, CPC), :], idx)

    @pl.when(c == 1)
    def _():
        pltpu.sync_copy(d_hbm.at[pl.ds(cb, CPC), :], idx)

    plsc.subcore_barrier()

    def body(i, carry):
        pltpu.sync_copy(ones_v, acc.at[idx.at[i]], add=True)
        return carry

    lax.fori_loop(0, CPC, body, 0)
    plsc.subcore_barrier()

    @pl.when(c == 0)
    def _():
        pltpu.sync_copy(acc.at[pl.ds(zb, PT)], outs_hbm.at[pl.ds(zb, PT)])

    @pl.when(c == 1)
    def _():
        pltpu.sync_copy(acc.at[pl.ds(zb, PT)], outd_hbm.at[pl.ds(zb, PT)])


def _make_scatter(k):
    @functools.partial(
        pl.kernel,
        out_type=jax.ShapeDtypeStruct((NC, NPAD, k), jnp.float32),
        mesh=_mesh,
        scratch_types=[
            pltpu.VMEM_SHARED((NPAD, k), jnp.float32),
            pltpu.VMEM((4, 2, CS), jnp.int32),
            pltpu.VMEM((CPW, CS), jnp.int32),
            pltpu.VMEM((2, CS, k), jnp.float32),
            pltpu.SemaphoreType.DMA,
            pltpu.SemaphoreType.DMA,
            pltpu.SemaphoreType.DMA,
            pltpu.SemaphoreType.DMA,
            pltpu.SemaphoreType.DMA,
            pltpu.SemaphoreType.DMA,
        ],
    )
    def _sc_scatter(tab_hbm, g_hbm, s_hbm, z_hbm, out_hbm,
                    acc, idx_g, idx_s, rows, sem0, sem1, si0, si1, si2, si3):
        c = lax.axis_index("c")
        t = lax.axis_index("s")
        zb = t * PT
        pltpu.sync_copy(z_hbm.at[pl.ds(zb, PT), :], acc.at[pl.ds(zb, PT), :])
        wst = (c * NS + t) * CPW
        pltpu.sync_copy(s_hbm.at[pl.ds(wst, CPW), :], idx_s)
        plsc.subcore_barrier()

        si = [si0, si1, si2, si3]

        pltpu.sync_copy(g_hbm.at[pl.ds(wst, 2), :], idx_g.at[3])
        pltpu.async_copy(tab_hbm.at[idx_g.at[3, 0]], rows.at[0], sem0)
        pltpu.async_copy(tab_hbm.at[idx_g.at[3, 1]], rows.at[1], sem1)
        pltpu.async_copy(g_hbm.at[pl.ds(wst + 2, 2), :], idx_g.at[0], si0)
        pltpu.async_copy(g_hbm.at[pl.ds(wst + 4, 2), :], idx_g.at[1], si1)
        pltpu.async_copy(g_hbm.at[pl.ds(wst + 6, 2), :], idx_g.at[2], si2)

        def _emit_pair(g, m, fire=True, load=True):
            if fire:
                pltpu.make_async_copy(g_hbm.at[pl.ds(wst + g + 2, 2), :],
                                      idx_g.at[m], si[m]).wait()
            pltpu.make_async_copy(tab_hbm.at[idx_g.at[0, 0]], rows.at[0], sem0).wait()
            pltpu.sync_copy(rows.at[0], acc.at[idx_s.at[g]], add=True)
            if fire:
                pltpu.async_copy(tab_hbm.at[idx_g.at[m, 0]], rows.at[0], sem0)
            pltpu.make_async_copy(tab_hbm.at[idx_g.at[0, 1]], rows.at[1], sem1).wait()
            pltpu.sync_copy(rows.at[1], acc.at[idx_s.at[g + 1]], add=True)
            if fire:
                pltpu.async_copy(tab_hbm.at[idx_g.at[m, 1]], rows.at[1], sem1)
            if load:
                pltpu.async_copy(g_hbm.at[pl.ds(wst + g + 8, 2), :],
                                 idx_g.at[(m + 3) % 4], si[(m + 3) % 4])

        def octet(r, carry):
            g = r * 8
            for pi in range(4):
                _emit_pair(g + 2 * pi, pi)
            return carry

        lax.fori_loop(0, (CPW - 8) // 8, octet, 0)
        _emit_pair(CPW - 8, 0, load=False)
        _emit_pair(CPW - 6, 1, load=False)
        _emit_pair(CPW - 4, 2, load=False)
        _emit_pair(CPW - 2, 3, fire=False, load=False)
        plsc.subcore_barrier()
        pltpu.sync_copy(acc.at[pl.ds(zb, PT), :], out_hbm.at[c, pl.ds(zb, PT), :])

    return _sc_scatter


_scatter128 = _make_scatter(128)


def _prep_body(deg_s, deg_d, x, ds_s_o, ds_d_o, xs_o):
    rs_s = lax.rsqrt(jnp.maximum(deg_s[...], 1.0))
    rs_d = lax.rsqrt(jnp.maximum(deg_d[...], 1.0))
    ds_s_o[...] = rs_s
    ds_d_o[...] = rs_d
    xs_o[0:N, :] = x[...] * rs_d
    xs_o[N:NPAD, :] = jnp.zeros((NPAD - N, F), jnp.float32)


_tc_prep = pl.pallas_call(
    _prep_body,
    out_shape=[
        jax.ShapeDtypeStruct((N, 1), jnp.float32),
        jax.ShapeDtypeStruct((N, 1), jnp.float32),
        jax.ShapeDtypeStruct((NPAD, F), jnp.float32),
    ],
)


def _layer0_body(x, bp, ds_s, wb0, w0, o):
    ab = (bp[0, 0:N, :] + bp[1, 0:N, :]) * ds_s[...]
    hb = jnp.dot(ab, wb0[...], preferred_element_type=jnp.float32)
    h1 = jnp.maximum(x[...], hb)
    o[0:N, :] = jnp.dot(h1 * ds_s[...], w0[...], preferred_element_type=jnp.float32)
    o[N:NPAD, :] = jnp.zeros((NPAD - N, F), jnp.float32)


_tc_layer0 = pl.pallas_call(
    _layer0_body,
    out_shape=jax.ShapeDtypeStruct((NPAD, F), jnp.float32),
)

BN_BN = 2000
BN_NB = 6


def _bnlayer_body(fp, bp, ds_s, ds_d, g, beta, bback, wb, w, o, sums, sumsq):
    p = pl.program_id(0)
    j = pl.program_id(1)
    rows = lax.broadcasted_iota(jnp.int32, (BN_BN, 1), 0) + j * BN_BN
    mask = rows < N
    h = (fp[0] + fp[1]) * ds_d[...]

    @pl.when(p == 0)
    def _():
        hm = jnp.where(mask, h, 0.0)

        @pl.when(j == 0)
        def _():
            sums[...] = jnp.zeros_like(sums)
            sumsq[...] = jnp.zeros_like(sumsq)

        sums[...] += jnp.sum(hm, axis=0, keepdims=True)
        sumsq[...] += jnp.sum(hm * hm, axis=0, keepdims=True)

    @pl.when(p == 1)
    def _():
        m = sums[...] / N
        v = sumsq[...] / N - m * m
        inv = lax.rsqrt(v + 1e-5)
        h1 = jnp.maximum((h - m) * inv * g[...] + beta[...], 0.0)
        ab = (bp[0] + bp[1]) * ds_s[...]
        hbk = jnp.dot(ab, wb[...], preferred_element_type=jnp.float32) + bback[...]
        h2 = jnp.where(mask, jnp.maximum(h1, hbk) * ds_s[...], 0.0)
        o[...] = jnp.dot(h2, w[...], preferred_element_type=jnp.float32)


def _make_bnlayer(ko):
    return pl.pallas_call(
        _bnlayer_body,
        grid=(2, BN_NB),
        in_specs=[
            pl.BlockSpec((NC, BN_BN, F), lambda p, j: (0, j, 0)),
            pl.BlockSpec((NC, BN_BN, F), lambda p, j: (0, j * p, 0)),
            pl.BlockSpec((BN_BN, 1), lambda p, j: (j, 0)),
            pl.BlockSpec((BN_BN, 1), lambda p, j: (j, 0)),
            pl.BlockSpec((1, F), lambda p, j: (0, 0)),
            pl.BlockSpec((1, F), lambda p, j: (0, 0)),
            pl.BlockSpec((1, F), lambda p, j: (0, 0)),
            pl.BlockSpec((F, F), lambda p, j: (0, 0)),
            pl.BlockSpec((F, ko), lambda p, j: (0, 0)),
        ],
        out_specs=pl.BlockSpec((BN_BN, ko), lambda p, j: (j, 0)),
        out_shape=jax.ShapeDtypeStruct((NPAD, ko), jnp.float32),
        scratch_shapes=[
            pltpu.VMEM((1, F), jnp.float32),
            pltpu.VMEM((1, F), jnp.float32),
        ],
    )


_tc_bnlayer128 = _make_bnlayer(128)


def _final_body(f2p, ds_d, b2p, o):
    o[...] = (f2p[0, 0:N, :] + f2p[1, 0:N, :]) * ds_d[...] + b2p[...]


_tc_final = pl.pallas_call(
    _final_body,
    out_shape=jax.ShapeDtypeStruct((N, 128), jnp.float32),
)


def kernel(x, edge_index, Wb0, Wb1, Wb2, bb2, W0, W1, W2, b2, g0, beta0, g1, beta1):
    s = edge_index[0]
    d = edge_index[1]
    pad = (jnp.arange(PADE, dtype=jnp.int32) % 128) + N
    sp = jnp.concatenate([s, pad]).reshape(ECH, CS)
    dp = jnp.concatenate([d, pad]).reshape(ECH, CS)
    z1 = jnp.zeros((NPAD,), jnp.float32)
    z128 = jnp.zeros((NPAD, 128), jnp.float32)
    w2p = jnp.pad(W2, ((0, 0), (0, 88)))
    b2p = jnp.pad(b2, (0, 88))[None, :]
    zb = jnp.zeros((1, F), jnp.float32)

    deg_s, deg_d = _sc_degrees(sp, dp, z1)
    ds_s, ds_d, xs = _tc_prep(deg_s[:N, None], deg_d[:N, None], x)
    bagg = _scatter128(xs, dp, sp, z128)
    tf0 = _tc_layer0(x, bagg, ds_s, Wb0, W0)
    f0 = _scatter128(tf0, sp, dp, z128)
    tf1 = _tc_bnlayer128(f0, bagg, ds_s, ds_d, g0[None, :], beta0[None, :], zb, Wb1, W1)
    f1 = _scatter128(tf1, sp, dp, z128)
    tf2 = _tc_bnlayer128(f1, bagg, ds_s, ds_d, g1[None, :], beta1[None, :], bb2[None, :], Wb2, w2p)
    f2 = _scatter128(tf2, sp, dp, z128)
    out = _tc_final(f2, ds_d, b2p)
    return out[:, :40]
